# Initial kernel scaffold; baseline (speedup 1.0000x reference)
import jax
import jax.numpy as jnp
from jax.experimental import pallas as pl


def kernel(x, edge_index, W1, b1, W2, b2, Wl, bl):
    raise NotImplementedError("write your pallas kernel here")


# speedup vs baseline: 43.1907x; 43.1907x over previous
import functools

import jax
import jax.numpy as jnp
from jax import lax
from jax.experimental import pallas as pl
from jax.experimental.pallas import tpu as pltpu
from jax.experimental.pallas import tpu_sc as plsc

N = 10000
NP = 10240
D_IN = 128
DH = 16
DOUT = 7

E_RAW = 320000
E_USED = E_RAW - 1
NTILES = 32
CB = 128
CHUNKS = (E_USED + NTILES * CB - 1) // (NTILES * CB)
EP = NTILES * CHUNKS * CB
RPT = NP // 16

_MESH = dict(core_axis_name="c", subcore_axis_name="s")


@functools.partial(
    pl.kernel,
    out_type=jax.ShapeDtypeStruct((2, NP), jnp.float32),
    mesh=plsc.VectorSubcoreMesh(**_MESH),
    compiler_params=pltpu.CompilerParams(use_tc_tiling_on_sc=False),
    scratch_types=[
        pltpu.VMEM((CHUNKS, CB), jnp.int32),
        pltpu.VMEM((CB,), jnp.float32),
        pltpu.VMEM((RPT,), jnp.float32),
        pltpu.VMEM_SHARED((NP,), jnp.float32),
    ],
)
def _sc_deg(dst_hbm, out_hbm, dst_v, ones_v, zero_v, acc_sh):
    c = lax.axis_index("c")
    s = lax.axis_index("s")
    wid = s * 2 + c

    for k in range(CB // 16):
        ones_v[pl.ds(k * 16, 16)] = jnp.full((16,), 1.0, jnp.float32)

    def zb(i, _):
        zero_v[pl.ds(i * 16, 16)] = jnp.zeros((16,), jnp.float32)
        return 0

    lax.fori_loop(0, RPT // 16, zb, 0)
    pltpu.sync_copy(zero_v, acc_sh.at[pl.ds(s * RPT, RPT)])
    pltpu.sync_copy(dst_hbm.at[wid], dst_v)
    plsc.subcore_barrier()

    def body(i, _):
        pltpu.sync_copy(ones_v, acc_sh.at[dst_v.at[i]], add=True)
        return 0

    lax.fori_loop(0, CHUNKS, body, 0)
    plsc.subcore_barrier()
    pltpu.sync_copy(acc_sh.at[pl.ds(s * RPT, RPT)],
                    out_hbm.at[c, pl.ds(s * RPT, RPT)])


@functools.partial(
    pl.kernel,
    out_type=jax.ShapeDtypeStruct((2, NP, DH), jnp.float32),
    mesh=plsc.VectorSubcoreMesh(**_MESH),
    compiler_params=pltpu.CompilerParams(use_tc_tiling_on_sc=False),
    scratch_types=[
        pltpu.VMEM((CHUNKS, CB), jnp.int32),
        pltpu.VMEM((CHUNKS, CB), jnp.int32),
        pltpu.VMEM((2, CB, DH), jnp.float32),
        pltpu.VMEM((RPT, DH), jnp.float32),
        pltpu.VMEM_SHARED((NP, DH), jnp.float32),
        pltpu.SemaphoreType.DMA,
        pltpu.SemaphoreType.DMA,
    ],
)
def _sc_agg(hs_hbm, src_hbm, dst_hbm, out_hbm,
            src_v, dst_v, rows_v, zero_v, acc_sh, sem0, sem1):
    c = lax.axis_index("c")
    s = lax.axis_index("s")
    wid = s * 2 + c

    def zb(i, _):
        zero_v[i] = jnp.zeros((DH,), jnp.float32)
        return 0

    lax.fori_loop(0, RPT, zb, 0)
    pltpu.sync_copy(zero_v, acc_sh.at[pl.ds(s * RPT, RPT)])
    pltpu.sync_copy(src_hbm.at[wid], src_v)
    pltpu.sync_copy(dst_hbm.at[wid], dst_v)
    plsc.subcore_barrier()

    sems = (sem0, sem1)
    pltpu.async_copy(hs_hbm.at[src_v.at[0]], rows_v.at[0], sems[0])

    def body(it, _):
        j = it * 2
        for b in range(2):
            jj = j + b
            nb = (b + 1) % 2

            @pl.when(jj < CHUNKS)
            def _process():
                @pl.when(jj + 1 < CHUNKS)
                def _prefetch():
                    pltpu.async_copy(hs_hbm.at[src_v.at[jj + 1]],
                                     rows_v.at[nb], sems[nb])

                pltpu.make_async_copy(hs_hbm.at[src_v.at[jj]],
                                      rows_v.at[b], sems[b]).wait()
                pltpu.sync_copy(rows_v.at[b], acc_sh.at[dst_v.at[jj]],
                                add=True)

        return 0

    lax.fori_loop(0, (CHUNKS + 1) // 2, body, 0)
    plsc.subcore_barrier()
    pltpu.sync_copy(acc_sh.at[pl.ds(s * RPT, RPT)],
                    out_hbm.at[c, pl.ds(s * RPT, RPT)])


BR = 1024


def _tc1_body(x_ref, w1_ref, p0_ref, p1_ref, hs_ref, dis_ref):
    deg = p0_ref[...] + p1_ref[...] + 1.0
    dis = lax.rsqrt(deg)
    h = jnp.dot(x_ref[...], w1_ref[...], preferred_element_type=jnp.float32)
    hs_ref[...] = h * dis
    dis_ref[...] = dis


_tc1 = pl.pallas_call(
    _tc1_body,
    grid=(NP // BR,),
    in_specs=[
        pl.BlockSpec((BR, D_IN), lambda i: (i, 0)),
        pl.BlockSpec((D_IN, DH), lambda i: (0, 0)),
        pl.BlockSpec((BR, 1), lambda i: (i, 0)),
        pl.BlockSpec((BR, 1), lambda i: (i, 0)),
    ],
    out_specs=[
        pl.BlockSpec((BR, DH), lambda i: (i, 0)),
        pl.BlockSpec((BR, 1), lambda i: (i, 0)),
    ],
    out_shape=[
        jax.ShapeDtypeStruct((NP, DH), jnp.float32),
        jax.ShapeDtypeStruct((NP, 1), jnp.float32),
    ],
)


def _tc2_body(p0_ref, p1_ref, hs1_ref, dis_ref, b1_ref, w2_ref, hs2_ref):
    dis = dis_ref[...]
    h2 = jnp.maximum(dis * (p0_ref[...] + p1_ref[...] + hs1_ref[...])
                     + b1_ref[...], 0.0)
    hs2_ref[...] = jnp.dot(h2, w2_ref[...],
                           preferred_element_type=jnp.float32) * dis


_tc2 = pl.pallas_call(
    _tc2_body,
    grid=(NP // BR,),
    in_specs=[
        pl.BlockSpec((BR, DH), lambda i: (i, 0)),
        pl.BlockSpec((BR, DH), lambda i: (i, 0)),
        pl.BlockSpec((BR, DH), lambda i: (i, 0)),
        pl.BlockSpec((BR, 1), lambda i: (i, 0)),
        pl.BlockSpec((1, DH), lambda i: (0, 0)),
        pl.BlockSpec((DH, DH), lambda i: (0, 0)),
    ],
    out_specs=pl.BlockSpec((BR, DH), lambda i: (i, 0)),
    out_shape=jax.ShapeDtypeStruct((NP, DH), jnp.float32),
)


def _tc3_body(p0_ref, p1_ref, hs2_ref, dis_ref, b2_ref, wl_ref, bl_ref,
              emb_ref, logp_ref):
    out2 = (dis_ref[...] * (p0_ref[...] + p1_ref[...] + hs2_ref[...])
            + b2_ref[...])
    emb_ref[...] = out2
    h3 = jnp.maximum(out2, 0.0)
    logits = jnp.dot(h3, wl_ref[...],
                     preferred_element_type=jnp.float32) + bl_ref[...]
    m = jnp.max(logits, axis=1, keepdims=True)
    e = jnp.exp(logits - m)
    lse = jnp.log(jnp.sum(e, axis=1, keepdims=True)) + m
    logp_ref[...] = logits - lse


_tc3 = pl.pallas_call(
    _tc3_body,
    grid=(NP // BR,),
    in_specs=[
        pl.BlockSpec((BR, DH), lambda i: (i, 0)),
        pl.BlockSpec((BR, DH), lambda i: (i, 0)),
        pl.BlockSpec((BR, DH), lambda i: (i, 0)),
        pl.BlockSpec((BR, 1), lambda i: (i, 0)),
        pl.BlockSpec((1, DH), lambda i: (0, 0)),
        pl.BlockSpec((DH, DOUT), lambda i: (0, 0)),
        pl.BlockSpec((1, DOUT), lambda i: (0, 0)),
    ],
    out_specs=[
        pl.BlockSpec((BR, DH), lambda i: (i, 0)),
        pl.BlockSpec((BR, DOUT), lambda i: (i, 0)),
    ],
    out_shape=[
        jax.ShapeDtypeStruct((NP, DH), jnp.float32),
        jax.ShapeDtypeStruct((NP, DOUT), jnp.float32),
    ],
)


@jax.jit
def kernel(x, edge_index, W1, b1, W2, b2, Wl, bl):
    ei = edge_index[:, 1:]
    pad = jnp.full((EP - E_USED,), N, dtype=jnp.int32)
    src_p = jnp.concatenate([ei[0], pad]).reshape(NTILES, CHUNKS, CB)
    dst_p = jnp.concatenate([ei[1], pad]).reshape(NTILES, CHUNKS, CB)
    xp = jnp.pad(x, ((0, NP - N), (0, 0)))

    degp = _sc_deg(dst_p)
    hs1, dis = _tc1(xp, W1,
                    degp[0].reshape(NP, 1), degp[1].reshape(NP, 1))
    agg1 = _sc_agg(hs1, src_p, dst_p)
    hs2 = _tc2(agg1[0], agg1[1], hs1, dis, b1.reshape(1, DH), W2)
    agg2 = _sc_agg(hs2, src_p, dst_p)
    emb, logp = _tc3(agg2[0], agg2[1], hs2, dis,
                     b2.reshape(1, DH), Wl, bl.reshape(1, DOUT))
    return logp[:N], emb[:N]


# speedup vs baseline: 48.2955x; 1.1182x over previous
import functools

import jax
import jax.numpy as jnp
from jax import lax
from jax.experimental import pallas as pl
from jax.experimental.pallas import tpu as pltpu
from jax.experimental.pallas import tpu_sc as plsc

N = 10000
NP = 10240
D_IN = 128
DH = 16
DOUT = 7

E_RAW = 320000
E_USED = E_RAW - 1
NTILES = 32
CB = 128
CHUNKS = (E_USED + NTILES * CB - 1) // (NTILES * CB)
EP = NTILES * CHUNKS * CB
RPT = NP // 16
NBUF = 8
LOOK = 4

_MESH = dict(core_axis_name="c", subcore_axis_name="s")


@functools.partial(
    pl.kernel,
    out_type=jax.ShapeDtypeStruct((2, NP), jnp.float32),
    mesh=plsc.VectorSubcoreMesh(**_MESH),
    compiler_params=pltpu.CompilerParams(use_tc_tiling_on_sc=False),
    scratch_types=[
        pltpu.VMEM((CHUNKS, CB), jnp.int32),
        pltpu.VMEM((CB,), jnp.float32),
        pltpu.VMEM((RPT,), jnp.float32),
        pltpu.VMEM_SHARED((NP,), jnp.float32),
        pltpu.SemaphoreType.DMA,
    ],
)
def _sc_deg(dst_hbm, out_hbm, dst_v, ones_v, zero_v, acc_sh, sem_s):
    c = lax.axis_index("c")
    s = lax.axis_index("s")
    wid = s * 2 + c

    for k in range(CB // 16):
        ones_v[pl.ds(k * 16, 16)] = jnp.full((16,), 1.0, jnp.float32)

    def zb(i, _):
        zero_v[pl.ds(i * 16, 16)] = jnp.zeros((16,), jnp.float32)
        return 0

    lax.fori_loop(0, RPT // 16, zb, 0)
    pltpu.sync_copy(zero_v, acc_sh.at[pl.ds(s * RPT, RPT)])
    pltpu.sync_copy(dst_hbm.at[wid], dst_v)
    plsc.subcore_barrier()

    def body(i, _):
        pltpu.async_copy(ones_v, acc_sh.at[dst_v.at[i]], sem_s, add=True)
        return 0

    lax.fori_loop(0, CHUNKS, body, 0)

    def drain(i, _):
        pltpu.make_async_copy(ones_v, acc_sh.at[dst_v.at[i]], sem_s).wait()
        return 0

    lax.fori_loop(0, CHUNKS, drain, 0)
    plsc.subcore_barrier()
    pltpu.sync_copy(acc_sh.at[pl.ds(s * RPT, RPT)],
                    out_hbm.at[c, pl.ds(s * RPT, RPT)])


@functools.partial(
    pl.kernel,
    out_type=jax.ShapeDtypeStruct((2, NP, DH), jnp.float32),
    mesh=plsc.VectorSubcoreMesh(**_MESH),
    compiler_params=pltpu.CompilerParams(use_tc_tiling_on_sc=False),
    scratch_types=[
        pltpu.VMEM((CHUNKS, CB), jnp.int32),
        pltpu.VMEM((CHUNKS, CB), jnp.int32),
        pltpu.VMEM((NBUF, CB, DH), jnp.float32),
        pltpu.VMEM((RPT, DH), jnp.float32),
        pltpu.VMEM_SHARED((NP, DH), jnp.float32),
        [pltpu.SemaphoreType.DMA] * NBUF,
        [pltpu.SemaphoreType.DMA] * NBUF,
    ],
)
def _sc_agg(hs_hbm, src_hbm, dst_hbm, out_hbm,
            src_v, dst_v, rows_v, zero_v, acc_sh, sem_g, sem_s):
    c = lax.axis_index("c")
    s = lax.axis_index("s")
    wid = s * 2 + c

    def zb(i, _):
        zero_v[i] = jnp.zeros((DH,), jnp.float32)
        return 0

    lax.fori_loop(0, RPT, zb, 0)
    pltpu.sync_copy(zero_v, acc_sh.at[pl.ds(s * RPT, RPT)])
    pltpu.sync_copy(src_hbm.at[wid], src_v)
    pltpu.sync_copy(dst_hbm.at[wid], dst_v)
    plsc.subcore_barrier()

    def gather(j, b):
        return pltpu.async_copy(hs_hbm.at[src_v.at[j]], rows_v.at[b],
                                sem_g[b])

    def scat(j, b, issue):
        if issue:
            return pltpu.async_copy(rows_v.at[b], acc_sh.at[dst_v.at[j]],
                                    sem_s[b], add=True)
        return pltpu.make_async_copy(rows_v.at[b], acc_sh.at[dst_v.at[j]],
                                     sem_s[b])

    for j in range(LOOK):
        gather(j, j % NBUF)

    def body(blk, _):
        base = blk * NBUF
        for p in range(NBUF):
            jj = base + p

            @pl.when(jj < CHUNKS)
            def _step():
                @pl.when(jj >= NBUF - LOOK)
                def _free():
                    scat(jj - (NBUF - LOOK), (p + LOOK) % NBUF,
                         False).wait()

                @pl.when(jj + LOOK < CHUNKS)
                def _prefetch():
                    gather(jj + LOOK, (p + LOOK) % NBUF)

                pltpu.make_async_copy(hs_hbm.at[src_v.at[jj]],
                                      rows_v.at[p], sem_g[p]).wait()
                scat(jj, p, True)

        return 0

    lax.fori_loop(0, (CHUNKS + NBUF - 1) // NBUF, body, 0)
    for j in range(max(0, CHUNKS - (NBUF - LOOK)), CHUNKS):
        scat(j, j % NBUF, False).wait()
    plsc.subcore_barrier()
    pltpu.sync_copy(acc_sh.at[pl.ds(s * RPT, RPT)],
                    out_hbm.at[c, pl.ds(s * RPT, RPT)])


BR = 1024


def _tc1_body(x_ref, w1_ref, p0_ref, p1_ref, hs_ref, dis_ref):
    deg = p0_ref[...] + p1_ref[...] + 1.0
    dis = lax.rsqrt(deg)
    h = jnp.dot(x_ref[...], w1_ref[...], preferred_element_type=jnp.float32)
    hs_ref[...] = h * dis
    dis_ref[...] = dis


_tc1 = pl.pallas_call(
    _tc1_body,
    grid=(NP // BR,),
    in_specs=[
        pl.BlockSpec((BR, D_IN), lambda i: (i, 0)),
        pl.BlockSpec((D_IN, DH), lambda i: (0, 0)),
        pl.BlockSpec((BR, 1), lambda i: (i, 0)),
        pl.BlockSpec((BR, 1), lambda i: (i, 0)),
    ],
    out_specs=[
        pl.BlockSpec((BR, DH), lambda i: (i, 0)),
        pl.BlockSpec((BR, 1), lambda i: (i, 0)),
    ],
    out_shape=[
        jax.ShapeDtypeStruct((NP, DH), jnp.float32),
        jax.ShapeDtypeStruct((NP, 1), jnp.float32),
    ],
)


def _tc2_body(p0_ref, p1_ref, hs1_ref, dis_ref, b1_ref, w2_ref, hs2_ref):
    dis = dis_ref[...]
    h2 = jnp.maximum(dis * (p0_ref[...] + p1_ref[...] + hs1_ref[...])
                     + b1_ref[...], 0.0)
    hs2_ref[...] = jnp.dot(h2, w2_ref[...],
                           preferred_element_type=jnp.float32) * dis


_tc2 = pl.pallas_call(
    _tc2_body,
    grid=(NP // BR,),
    in_specs=[
        pl.BlockSpec((BR, DH), lambda i: (i, 0)),
        pl.BlockSpec((BR, DH), lambda i: (i, 0)),
        pl.BlockSpec((BR, DH), lambda i: (i, 0)),
        pl.BlockSpec((BR, 1), lambda i: (i, 0)),
        pl.BlockSpec((1, DH), lambda i: (0, 0)),
        pl.BlockSpec((DH, DH), lambda i: (0, 0)),
    ],
    out_specs=pl.BlockSpec((BR, DH), lambda i: (i, 0)),
    out_shape=jax.ShapeDtypeStruct((NP, DH), jnp.float32),
)


def _tc3_body(p0_ref, p1_ref, hs2_ref, dis_ref, b2_ref, wl_ref, bl_ref,
              emb_ref, logp_ref):
    out2 = (dis_ref[...] * (p0_ref[...] + p1_ref[...] + hs2_ref[...])
            + b2_ref[...])
    emb_ref[...] = out2
    h3 = jnp.maximum(out2, 0.0)
    logits = jnp.dot(h3, wl_ref[...],
                     preferred_element_type=jnp.float32) + bl_ref[...]
    m = jnp.max(logits, axis=1, keepdims=True)
    e = jnp.exp(logits - m)
    lse = jnp.log(jnp.sum(e, axis=1, keepdims=True)) + m
    logp_ref[...] = logits - lse


_tc3 = pl.pallas_call(
    _tc3_body,
    grid=(NP // BR,),
    in_specs=[
        pl.BlockSpec((BR, DH), lambda i: (i, 0)),
        pl.BlockSpec((BR, DH), lambda i: (i, 0)),
        pl.BlockSpec((BR, DH), lambda i: (i, 0)),
        pl.BlockSpec((BR, 1), lambda i: (i, 0)),
        pl.BlockSpec((1, DH), lambda i: (0, 0)),
        pl.BlockSpec((DH, DOUT), lambda i: (0, 0)),
        pl.BlockSpec((1, DOUT), lambda i: (0, 0)),
    ],
    out_specs=[
        pl.BlockSpec((BR, DH), lambda i: (i, 0)),
        pl.BlockSpec((BR, DOUT), lambda i: (i, 0)),
    ],
    out_shape=[
        jax.ShapeDtypeStruct((NP, DH), jnp.float32),
        jax.ShapeDtypeStruct((NP, DOUT), jnp.float32),
    ],
)


@jax.jit
def kernel(x, edge_index, W1, b1, W2, b2, Wl, bl):
    ei = edge_index[:, 1:]
    pad = jnp.full((EP - E_USED,), N, dtype=jnp.int32)
    src_p = jnp.concatenate([ei[0], pad]).reshape(NTILES, CHUNKS, CB)
    dst_p = jnp.concatenate([ei[1], pad]).reshape(NTILES, CHUNKS, CB)
    xp = jnp.pad(x, ((0, NP - N), (0, 0)))

    degp = _sc_deg(dst_p)
    hs1, dis = _tc1(xp, W1,
                    degp[0].reshape(NP, 1), degp[1].reshape(NP, 1))
    agg1 = _sc_agg(hs1, src_p, dst_p)
    hs2 = _tc2(agg1[0], agg1[1], hs1, dis, b1.reshape(1, DH), W2)
    agg2 = _sc_agg(hs2, src_p, dst_p)
    emb, logp = _tc3(agg2[0], agg2[1], hs2, dis,
                     b2.reshape(1, DH), Wl, bl.reshape(1, DOUT))
    return logp[:N], emb[:N]


# speedup vs baseline: 67.4022x; 1.3956x over previous
import functools

import jax
import jax.numpy as jnp
from jax import lax
from jax.experimental import pallas as pl
from jax.experimental.pallas import tpu as pltpu
from jax.experimental.pallas import tpu_sc as plsc

N = 10000
NA = 10240
D_IN = 128
DH = 16
DOUT = 7

E_RAW = 320000
NTILES = 32
CB = 125
CHUNKS = 80
RPT = N // 16
RPTA = NA // 16
NBUF = 8
LOOK = 4
DEAD = N + 16

_MESH = dict(core_axis_name="c", subcore_axis_name="s")


def _redirect_edge0(idx_v, val):
    lane = lax.iota(jnp.int32, 16)
    row = idx_v[0, pl.ds(0, 16)]
    idx_v[0, pl.ds(0, 16)] = jnp.where(lane == 0, val, row)


@functools.partial(
    pl.kernel,
    out_type=jax.ShapeDtypeStruct((2, NA), jnp.float32),
    mesh=plsc.VectorSubcoreMesh(**_MESH),
    compiler_params=pltpu.CompilerParams(use_tc_tiling_on_sc=False),
    scratch_types=[
        pltpu.VMEM((CHUNKS, CB), jnp.int32),
        pltpu.VMEM((CB,), jnp.float32),
        pltpu.VMEM((RPTA,), jnp.float32),
        pltpu.VMEM_SHARED((NA,), jnp.float32),
        pltpu.SemaphoreType.DMA,
    ],
)
def _sc_deg(ei_hbm, out_hbm, dst_v, ones_v, zero_v, acc_sh, sem_s):
    c = lax.axis_index("c")
    s = lax.axis_index("s")
    wid = s * 2 + c

    for k in range(CB // 16 + 1):
        o = min(k * 16, CB - 16)
        ones_v[pl.ds(o, 16)] = jnp.full((16,), 1.0, jnp.float32)

    def zb(i, _):
        zero_v[pl.ds(i * 16, 16)] = jnp.zeros((16,), jnp.float32)
        return 0

    lax.fori_loop(0, RPTA // 16, zb, 0)
    pltpu.sync_copy(zero_v, acc_sh.at[pl.ds(s * RPTA, RPTA)])
    pltpu.sync_copy(ei_hbm.at[1, wid], dst_v)

    @pl.when(wid == 0)
    def _():
        _redirect_edge0(dst_v, DEAD)

    plsc.subcore_barrier()

    def body(i, _):
        pltpu.async_copy(ones_v, acc_sh.at[dst_v.at[i]], sem_s, add=True)
        return 0

    lax.fori_loop(0, CHUNKS, body, 0)

    def drain(i, _):
        pltpu.make_async_copy(ones_v, acc_sh.at[dst_v.at[i]], sem_s).wait()
        return 0

    lax.fori_loop(0, CHUNKS, drain, 0)
    plsc.subcore_barrier()
    pltpu.sync_copy(acc_sh.at[pl.ds(s * RPTA, RPTA)],
                    out_hbm.at[c, pl.ds(s * RPTA, RPTA)])


@functools.partial(
    pl.kernel,
    out_type=jax.ShapeDtypeStruct((2, N, DH), jnp.float32),
    mesh=plsc.VectorSubcoreMesh(**_MESH),
    compiler_params=pltpu.CompilerParams(use_tc_tiling_on_sc=False),
    scratch_types=[
        pltpu.VMEM((CHUNKS, CB), jnp.int32),
        pltpu.VMEM((CHUNKS, CB), jnp.int32),
        pltpu.VMEM((NBUF, CB, DH), jnp.float32),
        pltpu.VMEM((RPT, DH), jnp.float32),
        pltpu.VMEM_SHARED((NA, DH), jnp.float32),
        [pltpu.SemaphoreType.DMA] * NBUF,
        [pltpu.SemaphoreType.DMA] * NBUF,
    ],
)
def _sc_agg(hs_hbm, ei_hbm, out_hbm,
            src_v, dst_v, rows_v, zero_v, acc_sh, sem_g, sem_s):
    c = lax.axis_index("c")
    s = lax.axis_index("s")
    wid = s * 2 + c

    def zb(i, _):
        zero_v[i] = jnp.zeros((DH,), jnp.float32)
        return 0

    lax.fori_loop(0, RPT, zb, 0)
    pltpu.sync_copy(zero_v, acc_sh.at[pl.ds(s * RPTA, RPT)])
    pltpu.sync_copy(zero_v.at[pl.ds(0, RPTA - RPT)],
                    acc_sh.at[pl.ds(s * RPTA + RPT, RPTA - RPT)])
    pltpu.sync_copy(ei_hbm.at[0, wid], src_v)
    pltpu.sync_copy(ei_hbm.at[1, wid], dst_v)

    @pl.when(wid == 0)
    def _():
        _redirect_edge0(src_v, 0)
        _redirect_edge0(dst_v, DEAD)

    plsc.subcore_barrier()

    def gather(j, b):
        return pltpu.async_copy(hs_hbm.at[src_v.at[j]], rows_v.at[b],
                                sem_g[b])

    def scat(j, b, issue):
        if issue:
            return pltpu.async_copy(rows_v.at[b], acc_sh.at[dst_v.at[j]],
                                    sem_s[b], add=True)
        return pltpu.make_async_copy(rows_v.at[b], acc_sh.at[dst_v.at[j]],
                                     sem_s[b])

    for j in range(LOOK):
        gather(j, j % NBUF)

    def body(blk, _):
        base = blk * NBUF
        for p in range(NBUF):
            jj = base + p

            @pl.when(jj < CHUNKS)
            def _step():
                @pl.when(jj >= NBUF - LOOK)
                def _free():
                    scat(jj - (NBUF - LOOK), (p + LOOK) % NBUF,
                         False).wait()

                @pl.when(jj + LOOK < CHUNKS)
                def _prefetch():
                    gather(jj + LOOK, (p + LOOK) % NBUF)

                pltpu.make_async_copy(hs_hbm.at[src_v.at[jj]],
                                      rows_v.at[p], sem_g[p]).wait()
                scat(jj, p, True)

        return 0

    lax.fori_loop(0, (CHUNKS + NBUF - 1) // NBUF, body, 0)
    for j in range(max(0, CHUNKS - (NBUF - LOOK)), CHUNKS):
        scat(j, j % NBUF, False).wait()
    plsc.subcore_barrier()
    pltpu.sync_copy(acc_sh.at[pl.ds(s * RPT, RPT)],
                    out_hbm.at[c, pl.ds(s * RPT, RPT)])


BR = 1000


def _tc1_body(x_ref, w1_ref, pa_ref, pb_ref, hs_ref, dis_ref):
    p0 = jnp.reshape(pa_ref[...], (BR, 1))
    p1 = jnp.reshape(pb_ref[...], (BR, 1))
    dis = lax.rsqrt(p0 + p1 + 1.0)
    h = jnp.dot(x_ref[...], w1_ref[...], preferred_element_type=jnp.float32)
    hs_ref[...] = h * dis
    dis_ref[...] = dis


_tc1 = pl.pallas_call(
    _tc1_body,
    grid=(N // BR,),
    in_specs=[
        pl.BlockSpec((BR, D_IN), lambda i: (i, 0)),
        pl.BlockSpec((D_IN, DH), lambda i: (0, 0)),
        pl.BlockSpec((1, BR, 1), lambda i: (0, i, 0)),
        pl.BlockSpec((1, BR, 1), lambda i: (1, i, 0)),
    ],
    out_specs=[
        pl.BlockSpec((BR, DH), lambda i: (i, 0)),
        pl.BlockSpec((BR, 1), lambda i: (i, 0)),
    ],
    out_shape=[
        jax.ShapeDtypeStruct((N, DH), jnp.float32),
        jax.ShapeDtypeStruct((N, 1), jnp.float32),
    ],
)


def _tc2_body(pa_ref, pb_ref, hs1_ref, dis_ref, b1_ref, w2_ref, hs2_ref):
    p0 = jnp.reshape(pa_ref[...], (BR, DH))
    p1 = jnp.reshape(pb_ref[...], (BR, DH))
    dis = dis_ref[...]
    h2 = jnp.maximum(dis * (p0 + p1 + hs1_ref[...]) + b1_ref[...], 0.0)
    hs2_ref[...] = jnp.dot(h2, w2_ref[...],
                           preferred_element_type=jnp.float32) * dis


_tc2 = pl.pallas_call(
    _tc2_body,
    grid=(N // BR,),
    in_specs=[
        pl.BlockSpec((1, BR, DH), lambda i: (0, i, 0)),
        pl.BlockSpec((1, BR, DH), lambda i: (1, i, 0)),
        pl.BlockSpec((BR, DH), lambda i: (i, 0)),
        pl.BlockSpec((BR, 1), lambda i: (i, 0)),
        pl.BlockSpec((1, DH), lambda i: (0, 0)),
        pl.BlockSpec((DH, DH), lambda i: (0, 0)),
    ],
    out_specs=pl.BlockSpec((BR, DH), lambda i: (i, 0)),
    out_shape=jax.ShapeDtypeStruct((N, DH), jnp.float32),
)


def _tc3_body(pa_ref, pb_ref, hs2_ref, dis_ref, b2_ref, wl_ref, bl_ref,
              emb_ref, logp_ref):
    p0 = jnp.reshape(pa_ref[...], (BR, DH))
    p1 = jnp.reshape(pb_ref[...], (BR, DH))
    out2 = dis_ref[...] * (p0 + p1 + hs2_ref[...]) + b2_ref[...]
    emb_ref[...] = out2
    h3 = jnp.maximum(out2, 0.0)
    logits = jnp.dot(h3, wl_ref[...],
                     preferred_element_type=jnp.float32) + bl_ref[...]
    m = jnp.max(logits, axis=1, keepdims=True)
    e = jnp.exp(logits - m)
    lse = jnp.log(jnp.sum(e, axis=1, keepdims=True)) + m
    logp_ref[...] = logits - lse


_tc3 = pl.pallas_call(
    _tc3_body,
    grid=(N // BR,),
    in_specs=[
        pl.BlockSpec((1, BR, DH), lambda i: (0, i, 0)),
        pl.BlockSpec((1, BR, DH), lambda i: (1, i, 0)),
        pl.BlockSpec((BR, DH), lambda i: (i, 0)),
        pl.BlockSpec((BR, 1), lambda i: (i, 0)),
        pl.BlockSpec((1, DH), lambda i: (0, 0)),
        pl.BlockSpec((DH, DOUT), lambda i: (0, 0)),
        pl.BlockSpec((1, DOUT), lambda i: (0, 0)),
    ],
    out_specs=[
        pl.BlockSpec((BR, DH), lambda i: (i, 0)),
        pl.BlockSpec((BR, DOUT), lambda i: (i, 0)),
    ],
    out_shape=[
        jax.ShapeDtypeStruct((N, DH), jnp.float32),
        jax.ShapeDtypeStruct((N, DOUT), jnp.float32),
    ],
)


@jax.jit
def kernel(x, edge_index, W1, b1, W2, b2, Wl, bl):
    ei4 = edge_index.reshape(2, NTILES, CHUNKS, CB)

    degp = _sc_deg(ei4).reshape(2, NA, 1)
    hs1, dis = _tc1(x, W1, degp, degp)
    agg1 = _sc_agg(hs1, ei4)
    hs2 = _tc2(agg1, agg1, hs1, dis, b1.reshape(1, DH), W2)
    agg2 = _sc_agg(hs2, ei4)
    emb, logp = _tc3(agg2, agg2, hs2, dis,
                     b2.reshape(1, DH), Wl, bl.reshape(1, DOUT))
    return logp, emb
